# baseline (speedup 1.0000x reference)
import functools

import jax
import jax.numpy as jnp
from jax import lax
from jax.experimental import pallas as pl
from jax.experimental.pallas import tpu as pltpu
from jax.experimental.pallas import tpu_sc as plsc

_NC = 2
_NS = 16
_NW = _NC * _NS
_LANES = 128
_SUBL = 8


def _gather_call(n_rows, n_cols, n_table, block_cols):
    n_strips = n_rows // _SUBL
    blocks_per_strip = n_cols // block_cols
    n_blocks = n_strips * blocks_per_strip
    blocks_per_w = n_blocks // _NW
    segs = block_cols // _LANES
    stage = 8000
    n_stage = n_table // stage
    mesh = plsc.VectorSubcoreMesh(core_axis_name="c", subcore_axis_name="s")

    @functools.partial(
        pl.kernel,
        mesh=mesh,
        out_type=jax.ShapeDtypeStruct((n_rows, n_cols), jnp.float32),
        scratch_types=[
            pltpu.VMEM_SHARED((n_table,), jnp.float32),
            pltpu.VMEM((stage,), jnp.float32),
            pltpu.VMEM((_SUBL, block_cols), jnp.int32),
            pltpu.VMEM((_SUBL, block_cols), jnp.float32),
            pltpu.SemaphoreType.DMA,
        ],
    )
    def k(v_hbm, idx2d_hbm, out2d_hbm, tab_sp, stg_v, idx_v, val_v, sem):
        cid = lax.axis_index("c")
        sid = lax.axis_index("s")
        wid = sid * _NC + cid

        def stage_body(j, carry):
            c = sid + j * _NS

            @pl.when(c < n_stage)
            def _():
                off = c * stage
                pltpu.sync_copy(v_hbm.at[pl.ds(off, stage)], stg_v)
                pltpu.sync_copy(stg_v, tab_sp.at[pl.ds(off, stage)])

            return carry

        lax.fori_loop(0, (n_stage + _NS - 1) // _NS, stage_body, 0)

        plsc.subcore_barrier()

        idx_hbm = idx2d_hbm.reshape(n_strips, _SUBL, n_cols)
        out_hbm = out2d_hbm.reshape(n_strips, _SUBL, n_cols)

        base = wid * blocks_per_w

        def body(i, carry):
            q = base + i
            t = q // blocks_per_strip
            b = q % blocks_per_strip
            col0 = b * block_cols
            pltpu.sync_copy(
                idx_hbm.at[t, :, pl.ds(col0, block_cols)], idx_v
            )
            descs = []
            n_hbm = _SUBL * segs
            for r in range(_SUBL):
                for s in range(segs):
                    src = v_hbm if (r * segs + s) < n_hbm else tab_sp
                    descs.append(pltpu.async_copy(
                        src.at[idx_v.at[r, pl.ds(s * _LANES, _LANES)]],
                        val_v.at[r, pl.ds(s * _LANES, _LANES)],
                        sem,
                    ))
            for d in descs:
                d.wait()
            pltpu.sync_copy(
                val_v, out_hbm.at[t, :, pl.ds(col0, block_cols)]
            )
            return carry

        lax.fori_loop(0, blocks_per_w, body, 0)

    return k


def kernel(v, idx):
    b, s = idx.shape
    out_t = _gather_call(s, b, v.shape[0], 512)(v, idx.T.astype(jnp.int32))
    return out_t.T


# speedup vs baseline: 1.7392x; 1.7392x over previous
import functools

import jax
import jax.numpy as jnp
from jax import lax
from jax.experimental import pallas as pl
from jax.experimental.pallas import tpu as pltpu
from jax.experimental.pallas import tpu_sc as plsc

_NC = 2
_NS = 16
_NW = _NC * _NS
_LANES = 128
_SUBL = 8


def _gather_call(n_rows, n_cols, n_table, block_cols):
    n_strips = n_rows // _SUBL
    blocks_per_strip = n_cols // block_cols
    n_blocks = n_strips * blocks_per_strip
    blocks_per_w = n_blocks // _NW
    segs = block_cols // _LANES
    stage = 8000
    n_stage = n_table // stage
    mesh = plsc.VectorSubcoreMesh(core_axis_name="c", subcore_axis_name="s")

    @functools.partial(
        pl.kernel,
        mesh=mesh,
        out_type=jax.ShapeDtypeStruct((n_rows, n_cols), jnp.float32),
        scratch_types=[
            pltpu.VMEM_SHARED((n_table,), jnp.float32),
            pltpu.VMEM((stage,), jnp.float32),
            pltpu.VMEM((_SUBL, block_cols), jnp.int32),
            pltpu.VMEM((_SUBL, block_cols), jnp.float32),
            pltpu.SemaphoreType.DMA,
            pltpu.SemaphoreType.DMA,
        ],
    )
    def k(v_hbm, idx2d_hbm, out2d_hbm, tab_sp, stg_v, idx_v, val_v, sem,
          sem_h):
        cid = lax.axis_index("c")
        sid = lax.axis_index("s")
        wid = sid * _NC + cid

        def stage_body(j, carry):
            c = sid + j * _NS

            @pl.when(c < n_stage)
            def _():
                off = c * stage
                pltpu.sync_copy(v_hbm.at[pl.ds(off, stage)], stg_v)
                pltpu.sync_copy(stg_v, tab_sp.at[pl.ds(off, stage)])

            return carry

        lax.fori_loop(0, (n_stage + _NS - 1) // _NS, stage_body, 0)

        plsc.subcore_barrier()

        idx_hbm = idx2d_hbm.reshape(n_strips, _SUBL, n_cols)
        out_hbm = out2d_hbm.reshape(n_strips, _SUBL, n_cols)

        base = wid * blocks_per_w

        def body(i, carry):
            q = base + i
            t = q // blocks_per_strip
            b = q % blocks_per_strip
            col0 = b * block_cols
            pltpu.sync_copy(
                idx_hbm.at[t, :, pl.ds(col0, block_cols)], idx_v
            )
            descs = []
            n_hbm = 9
            for r in range(_SUBL):
                for s in range(segs):
                    hbm = (r * segs + s) < n_hbm
                    descs.append(pltpu.async_copy(
                        (v_hbm if hbm else tab_sp).at[
                            idx_v.at[r, pl.ds(s * _LANES, _LANES)]],
                        val_v.at[r, pl.ds(s * _LANES, _LANES)],
                        sem_h if hbm else sem,
                    ))
            for d in descs:
                d.wait()
            pltpu.sync_copy(
                val_v, out_hbm.at[t, :, pl.ds(col0, block_cols)]
            )
            return carry

        lax.fori_loop(0, blocks_per_w, body, 0)

    return k


def kernel(v, idx):
    b, s = idx.shape
    out_t = _gather_call(s, b, v.shape[0], 512)(v, idx.T.astype(jnp.int32))
    return out_t.T
